# baseline (speedup 1.0000x reference)
import functools

import jax
import jax.numpy as jnp
from jax import lax
from jax.experimental import pallas as pl
from jax.experimental.pallas import tpu as pltpu
from jax.experimental.pallas import tpu_sc as plsc

_NC = 2
_NS = 16
_NW = _NC * _NS


def _gather_rows(table, idx, chunk):
    V, Dp = table.shape
    N = idx.shape[0]
    assert N % _NW == 0
    n_w = N // _NW
    assert n_w % chunk == 0 and chunk % 8 == 0 and chunk <= 128
    steps = n_w // chunk
    mesh = plsc.VectorSubcoreMesh(core_axis_name="c", subcore_axis_name="s")

    @functools.partial(
        pl.kernel,
        mesh=mesh,
        compiler_params=pltpu.CompilerParams(use_tc_tiling_on_sc=False),
        out_type=jax.ShapeDtypeStruct((N, Dp), table.dtype),
        scratch_types=[
            pltpu.VMEM((chunk,), jnp.int32),
            pltpu.VMEM((chunk, Dp), table.dtype),
            pltpu.SemaphoreType.DMA,
        ],
    )
    def k(table_hbm, idx_hbm, out_hbm, idx_v, rows_v, sem):
        wid = lax.axis_index("s") * _NC + lax.axis_index("c")

        def body(s, carry):
            base = wid * n_w + s * chunk
            pltpu.sync_copy(idx_hbm.at[pl.ds(base, chunk)], idx_v)
            pltpu.async_copy(table_hbm.at[idx_v], rows_v, sem).wait()
            pltpu.sync_copy(rows_v, out_hbm.at[pl.ds(base, chunk)])
            return carry

        lax.fori_loop(0, steps, body, 0)

    return k(table, idx)


def _gather_rows2(table_a, table_b, idx, chunk):
    Va, Da = table_a.shape
    Vb, Db = table_b.shape
    N = idx.shape[0]
    n_w = N // _NW
    steps = n_w // chunk
    mesh = plsc.VectorSubcoreMesh(core_axis_name="c", subcore_axis_name="s")

    @functools.partial(
        pl.kernel,
        mesh=mesh,
        compiler_params=pltpu.CompilerParams(use_tc_tiling_on_sc=False),
        out_type=[jax.ShapeDtypeStruct((N, Da), table_a.dtype),
                  jax.ShapeDtypeStruct((N, Db), table_b.dtype)],
        scratch_types=[
            pltpu.VMEM((chunk,), jnp.int32),
            pltpu.VMEM((chunk, Da), table_a.dtype),
            pltpu.VMEM((chunk, Db), table_b.dtype),
            pltpu.SemaphoreType.DMA,
            pltpu.SemaphoreType.DMA,
        ],
    )
    def k(ta_hbm, tb_hbm, idx_hbm, out_a, out_b, idx_v, rows_a, rows_b,
          sem_a, sem_b):
        wid = lax.axis_index("s") * _NC + lax.axis_index("c")

        def body(s, carry):
            base = wid * n_w + s * chunk
            pltpu.sync_copy(idx_hbm.at[pl.ds(base, chunk)], idx_v)
            cp_a = pltpu.async_copy(ta_hbm.at[idx_v], rows_a, sem_a)
            cp_b = pltpu.async_copy(tb_hbm.at[idx_v], rows_b, sem_b)
            cp_a.wait()
            cp_b.wait()
            pltpu.sync_copy(rows_a, out_a.at[pl.ds(base, chunk)])
            pltpu.sync_copy(rows_b, out_b.at[pl.ds(base, chunk)])
            return carry

        lax.fori_loop(0, steps, body, 0)

    return k(table_a, table_b, idx)


def _stage_a_body(wi_ref, rat_ref, wu_ref, rt_ref,
                  w1a_ref, w1b_ref, b1_ref,
                  a1a_ref, a1b_ref, ab1_ref, a2w_ref, a2b_ref, a3_ref,
                  w2_ref, b2_ref, w3a_ref, w3b_ref, b3_ref, out_ref):
    LBU, D = wi_ref.shape
    BU = wu_ref.shape[0]
    L = LBU // BU
    relu = lambda x: jnp.maximum(x, 0.0)
    mm = lambda a, b: jnp.dot(a.astype(jnp.bfloat16), b.astype(jnp.bfloat16),
                              preferred_element_type=jnp.float32)
    wu = wu_ref[:]
    rtw1b = mm(rt_ref[:], w1b_ref[:])
    u_att = mm(wu, a1b_ref[:]) + ab1_ref[:]
    u_rep = jnp.broadcast_to(u_att[None], (L, BU, D)).reshape(LBU, D)
    iota8 = lax.broadcasted_iota(jnp.int32, (1, 8), 1)
    oh = (rat_ref[:] == iota8).astype(jnp.float32)
    x2 = relu(mm(wi_ref[:], w1a_ref[:]) + mm(oh, rtw1b) + b1_ref[:])
    h = relu(mm(x2, a1a_ref[:]) + u_rep)
    h = relu(mm(h, a2w_ref[:]) + a2b_ref[:])
    l2 = jnp.sum(h * a3_ref[:], axis=1, keepdims=True)
    l3 = l2.reshape(L, BU, 1)
    m = l3[0]
    for j in range(1, L):
        m = jnp.maximum(m, l3[j])
    e3 = jnp.exp(l3 - m[None])
    denom = e3[0]
    for j in range(1, L):
        denom = denom + e3[j]
    w3 = x2.reshape(L, BU, D) * e3
    acc = w3[0]
    for j in range(1, L):
        acc = acc + w3[j]
    hI = acc / denom
    hI = relu(mm(hI, w2_ref[:]) + b2_ref[:])
    out_ref[:] = relu(mm(wu, w3a_ref[:]) + mm(hI, w3b_ref[:]) + b3_ref[:])


def _stage_a(witem2, rat_col, wuser, rt_pad, w1a, w1b, b1,
             a1a, a1b, ab1, a2w, a2b, a3row, w2, b2, w3a, w3b, b3, BU):
    N, D = wuser.shape
    L = witem2.shape[0] // N
    grid = N // BU
    full = lambda arr: pl.BlockSpec(arr.shape, lambda i: (0,) * arr.ndim)
    return pl.pallas_call(
        _stage_a_body,
        grid=(grid,),
        in_specs=[
            pl.BlockSpec((L * BU, D), lambda i: (i, 0)),
            pl.BlockSpec((L * BU, 1), lambda i: (i, 0)),
            pl.BlockSpec((BU, D), lambda i: (i, 0)),
            full(rt_pad), full(w1a), full(w1b), full(b1),
            full(a1a), full(a1b), full(ab1), full(a2w), full(a2b), full(a3row),
            full(w2), full(b2), full(w3a), full(w3b), full(b3),
        ],
        out_specs=pl.BlockSpec((BU, D), lambda i: (i, 0)),
        out_shape=jax.ShapeDtypeStruct((N, D), jnp.float32),
    )(witem2, rat_col, wuser, rt_pad, w1a, w1b, b1,
      a1a, a1b, ab1, a2w, a2b, a3row, w2, b2, w3a, w3b, b3)


def _stage_b_body(f_ref, wu_ref,
                  sa1a_ref, sa1b_ref, sab1_ref, sa2w_ref, sa2b_ref, sa3_ref,
                  s1w_ref, s1b_ref, s2a_ref, s2b_ref, s2bias_ref,
                  s3w_ref, s3b_ref, out_ref):
    S = f_ref.shape[0] - 1
    relu = lambda x: jnp.maximum(x, 0.0)
    mm = lambda a, b: jnp.dot(a.astype(jnp.bfloat16), b.astype(jnp.bfloat16),
                              preferred_element_type=jnp.float32)
    wu = wu_ref[:]
    u_att = mm(wu, sa1b_ref[:]) + sab1_ref[:]
    sa2b = sa2b_ref[:]
    sa3 = sa3_ref[:]
    zs = []
    ls = []
    for s in range(S):
        z_s = f_ref[s + 1]
        a = relu(mm(z_s, sa1a_ref[:]) + u_att)
        a = relu(mm(a, sa2w_ref[:]) + sa2b)
        l_s = jnp.sum(a * sa3, axis=1, keepdims=True)
        zs.append(z_s)
        ls.append(l_s)
    m = ls[0]
    for s in range(1, S):
        m = jnp.maximum(m, ls[s])
    es = [jnp.exp(l_s - m) for l_s in ls]
    denom = es[0]
    for s in range(1, S):
        denom = denom + es[s]
    acc = zs[0] * es[0]
    for s in range(1, S):
        acc = acc + zs[s] * es[s]
    hS = acc / denom
    hS = relu(mm(hS, s1w_ref[:]) + s1b_ref[:])
    f = relu(mm(f_ref[0], s2a_ref[:]) + mm(hS, s2b_ref[:]) + s2bias_ref[:])
    out_ref[:] = relu(mm(f, s3w_ref[:]) + s3b_ref[:])


def _stage_b(feat3, wuser, sa1a, sa1b, sab1, sa2w, sa2b, sa3row,
             s1w, s1b, s2a, s2b, s2bias, s3w, s3b):
    S1, B, D = feat3.shape
    BN = 512
    grid = B // BN
    full = lambda arr: pl.BlockSpec(arr.shape, lambda i: (0,) * arr.ndim)
    return pl.pallas_call(
        _stage_b_body,
        grid=(grid,),
        in_specs=[
            pl.BlockSpec((S1, BN, D), lambda i: (0, i, 0)),
            pl.BlockSpec((BN, D), lambda i: (i, 0)),
            full(sa1a), full(sa1b), full(sab1), full(sa2w), full(sa2b),
            full(sa3row), full(s1w), full(s1b), full(s2a), full(s2b),
            full(s2bias), full(s3w), full(s3b),
        ],
        out_specs=pl.BlockSpec((BN, D), lambda i: (i, 0)),
        out_shape=jax.ShapeDtypeStruct((B, D), jnp.float32),
    )(feat3, wuser, sa1a, sa1b, sab1, sa2w, sa2b, sa3row,
      s1w, s1b, s2a, s2b, s2bias, s3w, s3b)


def kernel(nodes, item_history, itemrating_history, social_history,
           user_table, item_table, rating_table,
           i_ln1_w, i_ln1_b, i_ln2_w, i_ln2_b, i_ln3_w, i_ln3_b,
           ia1_w, ia1_b, ia2_w, ia2_b, ia3_w, ia3_b,
           s_ln1_w, s_ln1_b, s_ln2_w, s_ln2_b, s_ln3_w, s_ln3_b,
           sa1_w, sa1_b, sa2_w, sa2_b, sa3_w, sa3_b):
    NU, L = item_history.shape
    S = social_history.shape[1]
    D = user_table.shape[1]
    B = nodes.shape[0]
    i32 = jnp.int32
    f32 = jnp.float32
    nodes = nodes.astype(i32)

    hist_cat = jnp.concatenate(
        [item_history.astype(i32), itemrating_history.astype(i32),
         jnp.zeros((NU, 8), i32)], axis=1)
    social_pad = jnp.concatenate(
        [social_history.astype(i32), jnp.zeros((NU, 12), i32)], axis=1)

    soc = _gather_rows(social_pad, nodes, 32)[:, :S]
    u_all = jnp.concatenate([nodes, soc.T.reshape(-1)])
    NT = u_all.shape[0]

    hist_g, wuser_g = _gather_rows2(hist_cat, user_table.astype(f32),
                                    u_all, 112)

    BU = 512
    NB = NT // BU
    items_perm = (hist_g[:, :L].reshape(NB, BU, L)
                  .transpose(0, 2, 1).reshape(-1))
    rat_col = (hist_g[:, L:2 * L].reshape(NB, BU, L)
               .transpose(0, 2, 1).reshape(-1, 1))
    witem2 = _gather_rows(item_table.astype(f32), items_perm, 128)

    rt_pad = jnp.concatenate(
        [rating_table.astype(f32),
         jnp.zeros((8 - rating_table.shape[0], D), f32)], axis=0)
    row = lambda v: v.reshape(1, -1).astype(f32)
    feat = _stage_a(
        witem2, rat_col, wuser_g, rt_pad,
        i_ln1_w[:D], i_ln1_w[D:], row(i_ln1_b),
        ia1_w[:D], ia1_w[D:], row(ia1_b), ia2_w, row(ia2_b), row(ia3_w),
        i_ln2_w, row(i_ln2_b), i_ln3_w[:D], i_ln3_w[D:], row(i_ln3_b), BU)

    feat3 = feat.reshape(S + 1, B, D)
    return _stage_b(
        feat3, wuser_g,
        sa1_w[:D], sa1_w[D:], row(sa1_b), sa2_w, row(sa2_b), row(sa3_w),
        s_ln1_w, row(s_ln1_b), s_ln2_w[:D], s_ln2_w[D:], row(s_ln2_b),
        s_ln3_w, row(s_ln3_b))


# speedup vs baseline: 1.1912x; 1.1912x over previous
import functools

import jax
import jax.numpy as jnp
from jax import lax
from jax.experimental import pallas as pl
from jax.experimental.pallas import tpu as pltpu
from jax.experimental.pallas import tpu_sc as plsc

_NC = 2
_NS = 16
_NW = _NC * _NS


def _gather_rows(table, idx, chunk):
    V, Dp = table.shape
    N = idx.shape[0]
    assert N % _NW == 0
    n_w = N // _NW
    assert n_w % chunk == 0 and chunk % 8 == 0 and chunk <= 128
    steps = n_w // chunk
    mesh = plsc.VectorSubcoreMesh(core_axis_name="c", subcore_axis_name="s")

    @functools.partial(
        pl.kernel,
        mesh=mesh,
        compiler_params=pltpu.CompilerParams(use_tc_tiling_on_sc=False),
        out_type=jax.ShapeDtypeStruct((N, Dp), table.dtype),
        scratch_types=[
            pltpu.VMEM((chunk,), jnp.int32),
            pltpu.VMEM((chunk, Dp), table.dtype),
            pltpu.SemaphoreType.DMA,
        ],
    )
    def k(table_hbm, idx_hbm, out_hbm, idx_v, rows_v, sem):
        wid = lax.axis_index("s") * _NC + lax.axis_index("c")

        def body(s, carry):
            base = wid * n_w + s * chunk
            pltpu.sync_copy(idx_hbm.at[pl.ds(base, chunk)], idx_v)
            pltpu.async_copy(table_hbm.at[idx_v], rows_v, sem).wait()
            pltpu.sync_copy(rows_v, out_hbm.at[pl.ds(base, chunk)])
            return carry

        lax.fori_loop(0, steps, body, 0)

    return k(table, idx)


def _gather_rows_ring(table, idx, chunk):
    V, Dp = table.shape
    N = idx.shape[0]
    n_w = N // _NW
    steps = n_w // chunk
    assert steps % 2 == 0
    K = steps // 2
    mesh = plsc.VectorSubcoreMesh(core_axis_name="c", subcore_axis_name="s")

    @functools.partial(
        pl.kernel,
        mesh=mesh,
        compiler_params=pltpu.CompilerParams(use_tc_tiling_on_sc=False),
        out_type=jax.ShapeDtypeStruct((N, Dp), table.dtype),
        scratch_types=[
            pltpu.VMEM((chunk,), jnp.int32),
            pltpu.VMEM((chunk,), jnp.int32),
            pltpu.VMEM((chunk, Dp), table.dtype),
            pltpu.VMEM((chunk, Dp), table.dtype),
            pltpu.SemaphoreType.DMA,
            pltpu.SemaphoreType.DMA,
            pltpu.SemaphoreType.DMA,
            pltpu.SemaphoreType.DMA,
        ],
    )
    def k(table_hbm, idx_hbm, out_hbm, idx0, idx1, rows0, rows1,
          sg0, sg1, sw0, sw1):
        wid = lax.axis_index("s") * _NC + lax.axis_index("c")
        idx_v = (idx0, idx1)
        rows_v = (rows0, rows1)
        sg = (sg0, sg1)
        sw = (sw0, sw1)

        def sync_idx(b, c):
            base = wid * n_w + c * chunk
            pltpu.sync_copy(idx_hbm.at[pl.ds(base, chunk)], idx_v[b])

        def start_g(b):
            pltpu.async_copy(table_hbm.at[idx_v[b]], rows_v[b], sg[b])

        def wait_g(b):
            pltpu.make_async_copy(table_hbm.at[idx_v[b]], rows_v[b], sg[b]).wait()

        def start_w(b, c):
            base = wid * n_w + c * chunk
            pltpu.async_copy(rows_v[b], out_hbm.at[pl.ds(base, chunk)], sw[b])

        def wait_w(b, c):
            base = wid * n_w + c * chunk
            pltpu.make_async_copy(
                rows_v[b], out_hbm.at[pl.ds(base, chunk)], sw[b]).wait()

        sync_idx(0, 0)
        start_g(0)

        def body(kk, carry):
            c0 = 2 * kk
            wait_g(0)
            start_w(0, c0)
            sync_idx(1, c0 + 1)
            start_g(1)
            wait_g(1)
            start_w(1, c0 + 1)
            wait_w(0, c0)

            @pl.when(kk + 1 < K)
            def _():
                sync_idx(0, c0 + 2)
                start_g(0)

            wait_w(1, c0 + 1)
            return carry

        lax.fori_loop(0, K, body, 0)

    return k(table, idx)


def _gather_rows2(table_a, table_b, idx, chunk):
    Va, Da = table_a.shape
    Vb, Db = table_b.shape
    N = idx.shape[0]
    n_w = N // _NW
    steps = n_w // chunk
    mesh = plsc.VectorSubcoreMesh(core_axis_name="c", subcore_axis_name="s")

    @functools.partial(
        pl.kernel,
        mesh=mesh,
        compiler_params=pltpu.CompilerParams(use_tc_tiling_on_sc=False),
        out_type=[jax.ShapeDtypeStruct((N, Da), table_a.dtype),
                  jax.ShapeDtypeStruct((N, Db), table_b.dtype)],
        scratch_types=[
            pltpu.VMEM((chunk,), jnp.int32),
            pltpu.VMEM((chunk, Da), table_a.dtype),
            pltpu.VMEM((chunk, Db), table_b.dtype),
            pltpu.SemaphoreType.DMA,
            pltpu.SemaphoreType.DMA,
        ],
    )
    def k(ta_hbm, tb_hbm, idx_hbm, out_a, out_b, idx_v, rows_a, rows_b,
          sem_a, sem_b):
        wid = lax.axis_index("s") * _NC + lax.axis_index("c")

        def body(s, carry):
            base = wid * n_w + s * chunk
            pltpu.sync_copy(idx_hbm.at[pl.ds(base, chunk)], idx_v)
            cp_a = pltpu.async_copy(ta_hbm.at[idx_v], rows_a, sem_a)
            cp_b = pltpu.async_copy(tb_hbm.at[idx_v], rows_b, sem_b)
            cp_a.wait()
            cp_b.wait()
            pltpu.sync_copy(rows_a, out_a.at[pl.ds(base, chunk)])
            pltpu.sync_copy(rows_b, out_b.at[pl.ds(base, chunk)])
            return carry

        lax.fori_loop(0, steps, body, 0)

    return k(table_a, table_b, idx)


def _stage_a_body(wi_ref, hist_ref, wu_ref, rt_ref,
                  w1a_ref, w1b_ref, b1_ref,
                  a1a_ref, a1b_ref, ab1_ref, a2w_ref, a2b_ref, a3_ref,
                  w2_ref, b2_ref, w3a_ref, w3b_ref, b3_ref, out_ref):
    LB2, D2 = wi_ref.shape
    D = D2 // 2
    BU = wu_ref.shape[0]
    B2 = BU // 2
    L = LB2 // B2
    relu = lambda x: jnp.maximum(x, 0.0)
    mm = lambda a, b: jnp.dot(a.astype(jnp.bfloat16), b.astype(jnp.bfloat16),
                              preferred_element_type=jnp.float32)
    rtw1b = mm(rt_ref[:], w1b_ref[:])
    iota8 = lax.broadcasted_iota(jnp.int32, (1, 8), 1)
    for h in (0, 1):
        wu = wu_ref[h * B2:(h + 1) * B2]
        wi = wi_ref[:, h * D:(h + 1) * D]
        ohs = [(hist_ref[h * B2:(h + 1) * B2, L + j:L + j + 1] == iota8)
               .astype(jnp.float32) for j in range(L)]
        oh = jnp.concatenate(ohs, axis=0)
        u_att = mm(wu, a1b_ref[:]) + ab1_ref[:]
        u_rep = jnp.broadcast_to(u_att[None], (L, B2, D)).reshape(LB2, D)
        x2 = relu(mm(wi, w1a_ref[:]) + mm(oh, rtw1b) + b1_ref[:])
        hh = relu(mm(x2, a1a_ref[:]) + u_rep)
        hh = relu(mm(hh, a2w_ref[:]) + a2b_ref[:])
        l2 = jnp.sum(hh * a3_ref[:], axis=1, keepdims=True)
        l3 = l2.reshape(L, B2, 1)
        m = l3[0]
        for j in range(1, L):
            m = jnp.maximum(m, l3[j])
        e3 = jnp.exp(l3 - m[None])
        denom = e3[0]
        for j in range(1, L):
            denom = denom + e3[j]
        w3 = x2.reshape(L, B2, D) * e3
        acc = w3[0]
        for j in range(1, L):
            acc = acc + w3[j]
        hI = acc / denom
        hI = relu(mm(hI, w2_ref[:]) + b2_ref[:])
        out_ref[h * B2:(h + 1) * B2] = relu(
            mm(wu, w3a_ref[:]) + mm(hI, w3b_ref[:]) + b3_ref[:])


def _stage_a(witem_w, hist_g, wuser, rt_pad, w1a, w1b, b1,
             a1a, a1b, ab1, a2w, a2b, a3row, w2, b2, w3a, w3b, b3, BU):
    N, D = wuser.shape
    L = 2 * witem_w.shape[0] // N
    grid = N // BU
    full = lambda arr: pl.BlockSpec(arr.shape, lambda i: (0,) * arr.ndim)
    return pl.pallas_call(
        _stage_a_body,
        grid=(grid,),
        in_specs=[
            pl.BlockSpec((L * BU // 2, 2 * D), lambda i: (i, 0)),
            pl.BlockSpec((BU, 48), lambda i: (i, 0)),
            pl.BlockSpec((BU, D), lambda i: (i, 0)),
            full(rt_pad), full(w1a), full(w1b), full(b1),
            full(a1a), full(a1b), full(ab1), full(a2w), full(a2b), full(a3row),
            full(w2), full(b2), full(w3a), full(w3b), full(b3),
        ],
        out_specs=pl.BlockSpec((BU, D), lambda i: (i, 0)),
        out_shape=jax.ShapeDtypeStruct((N, D), jnp.float32),
    )(witem_w, hist_g, wuser, rt_pad, w1a, w1b, b1,
      a1a, a1b, ab1, a2w, a2b, a3row, w2, b2, w3a, w3b, b3)


def _stage_b_body(f_ref, wu_ref,
                  sa1a_ref, sa1b_ref, sab1_ref, sa2w_ref, sa2b_ref, sa3_ref,
                  s1w_ref, s1b_ref, s2a_ref, s2b_ref, s2bias_ref,
                  s3w_ref, s3b_ref, out_ref):
    S = f_ref.shape[0] - 1
    relu = lambda x: jnp.maximum(x, 0.0)
    mm = lambda a, b: jnp.dot(a.astype(jnp.bfloat16), b.astype(jnp.bfloat16),
                              preferred_element_type=jnp.float32)
    wu = wu_ref[:]
    u_att = mm(wu, sa1b_ref[:]) + sab1_ref[:]
    sa2b = sa2b_ref[:]
    sa3 = sa3_ref[:]
    zs = []
    ls = []
    for s in range(S):
        z_s = f_ref[s + 1]
        a = relu(mm(z_s, sa1a_ref[:]) + u_att)
        a = relu(mm(a, sa2w_ref[:]) + sa2b)
        l_s = jnp.sum(a * sa3, axis=1, keepdims=True)
        zs.append(z_s)
        ls.append(l_s)
    m = ls[0]
    for s in range(1, S):
        m = jnp.maximum(m, ls[s])
    es = [jnp.exp(l_s - m) for l_s in ls]
    denom = es[0]
    for s in range(1, S):
        denom = denom + es[s]
    acc = zs[0] * es[0]
    for s in range(1, S):
        acc = acc + zs[s] * es[s]
    hS = acc / denom
    hS = relu(mm(hS, s1w_ref[:]) + s1b_ref[:])
    f = relu(mm(f_ref[0], s2a_ref[:]) + mm(hS, s2b_ref[:]) + s2bias_ref[:])
    out_ref[:] = relu(mm(f, s3w_ref[:]) + s3b_ref[:])


def _stage_b(feat3, wuser, sa1a, sa1b, sab1, sa2w, sa2b, sa3row,
             s1w, s1b, s2a, s2b, s2bias, s3w, s3b):
    S1, B, D = feat3.shape
    BN = 512
    grid = B // BN
    full = lambda arr: pl.BlockSpec(arr.shape, lambda i: (0,) * arr.ndim)
    return pl.pallas_call(
        _stage_b_body,
        grid=(grid,),
        in_specs=[
            pl.BlockSpec((S1, BN, D), lambda i: (0, i, 0)),
            pl.BlockSpec((BN, D), lambda i: (i, 0)),
            full(sa1a), full(sa1b), full(sab1), full(sa2w), full(sa2b),
            full(sa3row), full(s1w), full(s1b), full(s2a), full(s2b),
            full(s2bias), full(s3w), full(s3b),
        ],
        out_specs=pl.BlockSpec((BN, D), lambda i: (i, 0)),
        out_shape=jax.ShapeDtypeStruct((B, D), jnp.float32),
    )(feat3, wuser, sa1a, sa1b, sab1, sa2w, sa2b, sa3row,
      s1w, s1b, s2a, s2b, s2bias, s3w, s3b)


def kernel(nodes, item_history, itemrating_history, social_history,
           user_table, item_table, rating_table,
           i_ln1_w, i_ln1_b, i_ln2_w, i_ln2_b, i_ln3_w, i_ln3_b,
           ia1_w, ia1_b, ia2_w, ia2_b, ia3_w, ia3_b,
           s_ln1_w, s_ln1_b, s_ln2_w, s_ln2_b, s_ln3_w, s_ln3_b,
           sa1_w, sa1_b, sa2_w, sa2_b, sa3_w, sa3_b):
    NU, L = item_history.shape
    S = social_history.shape[1]
    D = user_table.shape[1]
    B = nodes.shape[0]
    i32 = jnp.int32
    f32 = jnp.float32
    nodes = nodes.astype(i32)

    hist_cat = jnp.concatenate(
        [item_history.astype(i32), itemrating_history.astype(i32),
         jnp.zeros((NU, 8), i32)], axis=1)
    social_pad = jnp.concatenate(
        [social_history.astype(i32), jnp.zeros((NU, 12), i32)], axis=1)

    soc = _gather_rows(social_pad, nodes, 32)[:, :S]
    u_all = jnp.concatenate([nodes, soc.T.reshape(-1)])
    NT = u_all.shape[0]

    hist_g, wuser_g = _gather_rows2(hist_cat, user_table.astype(f32),
                                    u_all, 112)

    BU = 512
    NB = NT // BU
    B2 = BU // 2
    items_perm = (hist_g[:, :L].reshape(NB, 2, B2, L)
                  .transpose(0, 3, 2, 1).reshape(-1))
    witem = _gather_rows_ring(item_table.astype(f32), items_perm, 120)
    witem_w = witem.reshape(L * NT // 2, 2 * D)

    rt_pad = jnp.concatenate(
        [rating_table.astype(f32),
         jnp.zeros((8 - rating_table.shape[0], D), f32)], axis=0)
    row = lambda v: v.reshape(1, -1).astype(f32)
    feat = _stage_a(
        witem_w, hist_g, wuser_g, rt_pad,
        i_ln1_w[:D], i_ln1_w[D:], row(i_ln1_b),
        ia1_w[:D], ia1_w[D:], row(ia1_b), ia2_w, row(ia2_b), row(ia3_w),
        i_ln2_w, row(i_ln2_b), i_ln3_w[:D], i_ln3_w[D:], row(i_ln3_b), BU)

    feat3 = feat.reshape(S + 1, B, D)
    return _stage_b(
        feat3, wuser_g,
        sa1_w[:D], sa1_w[D:], row(sa1_b), sa2_w, row(sa2_b), row(sa3_w),
        s_ln1_w, row(s_ln1_b), s_ln2_w[:D], s_ln2_w[D:], row(s_ln2_b),
        s_ln3_w, row(s_ln3_b))


# speedup vs baseline: 1.2515x; 1.0506x over previous
import functools

import jax
import jax.numpy as jnp
from jax import lax
from jax.experimental import pallas as pl
from jax.experimental.pallas import tpu as pltpu
from jax.experimental.pallas import tpu_sc as plsc

_NC = 2
_NS = 16
_NW = _NC * _NS


def _gather_rows(table, idx, chunk):
    V, Dp = table.shape
    N = idx.shape[0]
    assert N % _NW == 0
    n_w = N // _NW
    assert n_w % chunk == 0 and chunk % 8 == 0 and chunk <= 128
    steps = n_w // chunk
    mesh = plsc.VectorSubcoreMesh(core_axis_name="c", subcore_axis_name="s")

    @functools.partial(
        pl.kernel,
        mesh=mesh,
        compiler_params=pltpu.CompilerParams(use_tc_tiling_on_sc=False),
        out_type=jax.ShapeDtypeStruct((N, Dp), table.dtype),
        scratch_types=[
            pltpu.VMEM((chunk,), jnp.int32),
            pltpu.VMEM((chunk, Dp), table.dtype),
            pltpu.SemaphoreType.DMA,
        ],
    )
    def k(table_hbm, idx_hbm, out_hbm, idx_v, rows_v, sem):
        wid = lax.axis_index("s") * _NC + lax.axis_index("c")

        def body(s, carry):
            base = wid * n_w + s * chunk
            pltpu.sync_copy(idx_hbm.at[pl.ds(base, chunk)], idx_v)
            pltpu.async_copy(table_hbm.at[idx_v], rows_v, sem).wait()
            pltpu.sync_copy(rows_v, out_hbm.at[pl.ds(base, chunk)])
            return carry

        lax.fori_loop(0, steps, body, 0)

    return k(table, idx)


def _gather_rows_ring(table, idx, chunk):
    V, Dp = table.shape
    N = idx.shape[0]
    n_w = N // _NW
    steps = n_w // chunk
    assert steps % 2 == 0
    K = steps // 2
    mesh = plsc.VectorSubcoreMesh(core_axis_name="c", subcore_axis_name="s")

    @functools.partial(
        pl.kernel,
        mesh=mesh,
        compiler_params=pltpu.CompilerParams(use_tc_tiling_on_sc=False),
        out_type=jax.ShapeDtypeStruct((N, Dp), table.dtype),
        scratch_types=[
            pltpu.VMEM((chunk,), jnp.int32),
            pltpu.VMEM((chunk,), jnp.int32),
            pltpu.VMEM((chunk, Dp), table.dtype),
            pltpu.VMEM((chunk, Dp), table.dtype),
            pltpu.SemaphoreType.DMA,
            pltpu.SemaphoreType.DMA,
            pltpu.SemaphoreType.DMA,
            pltpu.SemaphoreType.DMA,
        ],
    )
    def k(table_hbm, idx_hbm, out_hbm, idx0, idx1, rows0, rows1,
          sg0, sg1, sw0, sw1):
        wid = lax.axis_index("s") * _NC + lax.axis_index("c")
        idx_v = (idx0, idx1)
        rows_v = (rows0, rows1)
        sg = (sg0, sg1)
        sw = (sw0, sw1)

        def sync_idx(b, c):
            base = wid * n_w + c * chunk
            pltpu.sync_copy(idx_hbm.at[pl.ds(base, chunk)], idx_v[b])

        def start_g(b):
            pltpu.async_copy(table_hbm.at[idx_v[b]], rows_v[b], sg[b])

        def wait_g(b):
            pltpu.make_async_copy(table_hbm.at[idx_v[b]], rows_v[b], sg[b]).wait()

        def start_w(b, c):
            base = wid * n_w + c * chunk
            pltpu.async_copy(rows_v[b], out_hbm.at[pl.ds(base, chunk)], sw[b])

        def wait_w(b, c):
            base = wid * n_w + c * chunk
            pltpu.make_async_copy(
                rows_v[b], out_hbm.at[pl.ds(base, chunk)], sw[b]).wait()

        sync_idx(0, 0)
        start_g(0)

        def body(kk, carry):
            c0 = 2 * kk
            wait_g(0)
            start_w(0, c0)
            sync_idx(1, c0 + 1)
            start_g(1)
            wait_g(1)
            start_w(1, c0 + 1)
            wait_w(0, c0)

            @pl.when(kk + 1 < K)
            def _():
                sync_idx(0, c0 + 2)
                start_g(0)

            wait_w(1, c0 + 1)
            return carry

        lax.fori_loop(0, K, body, 0)

    return k(table, idx)


def _gather_rows2(table_a, table_b, idx, chunk):
    Va, Da = table_a.shape
    Vb, Db = table_b.shape
    N = idx.shape[0]
    n_w = N // _NW
    steps = n_w // chunk
    mesh = plsc.VectorSubcoreMesh(core_axis_name="c", subcore_axis_name="s")

    @functools.partial(
        pl.kernel,
        mesh=mesh,
        compiler_params=pltpu.CompilerParams(use_tc_tiling_on_sc=False),
        out_type=[jax.ShapeDtypeStruct((N, Da), table_a.dtype),
                  jax.ShapeDtypeStruct((N, Db), table_b.dtype)],
        scratch_types=[
            pltpu.VMEM((chunk,), jnp.int32),
            pltpu.VMEM((chunk, Da), table_a.dtype),
            pltpu.VMEM((chunk, Db), table_b.dtype),
            pltpu.SemaphoreType.DMA,
            pltpu.SemaphoreType.DMA,
        ],
    )
    def k(ta_hbm, tb_hbm, idx_hbm, out_a, out_b, idx_v, rows_a, rows_b,
          sem_a, sem_b):
        wid = lax.axis_index("s") * _NC + lax.axis_index("c")

        def body(s, carry):
            base = wid * n_w + s * chunk
            pltpu.sync_copy(idx_hbm.at[pl.ds(base, chunk)], idx_v)
            cp_a = pltpu.async_copy(ta_hbm.at[idx_v], rows_a, sem_a)
            cp_b = pltpu.async_copy(tb_hbm.at[idx_v], rows_b, sem_b)
            cp_a.wait()
            cp_b.wait()
            pltpu.sync_copy(rows_a, out_a.at[pl.ds(base, chunk)])
            pltpu.sync_copy(rows_b, out_b.at[pl.ds(base, chunk)])
            return carry

        lax.fori_loop(0, steps, body, 0)

    return k(table_a, table_b, idx)


def _stage_a_body(wi_ref, hist_ref, wu_ref, rt_ref,
                  w1a_ref, w1b_ref, b1_ref,
                  a1a_ref, a1b_ref, ab1_ref, a2w_ref, a2b_ref, a3_ref,
                  w2_ref, b2_ref, w3a_ref, w3b_ref, b3_ref, out_ref):
    LB2, D2 = wi_ref.shape
    D = D2 // 2
    BU = wu_ref.shape[0]
    B2 = BU // 2
    L = LB2 // B2
    relu = lambda x: jnp.maximum(x, 0.0)
    mm = lambda a, b: jnp.dot(a.astype(jnp.bfloat16), b.astype(jnp.bfloat16),
                              preferred_element_type=jnp.float32)
    rtw1b = mm(rt_ref[:], w1b_ref[:])
    iota8 = lax.broadcasted_iota(jnp.int32, (1, 8), 1)
    for h in (0, 1):
        wu = wu_ref[h * B2:(h + 1) * B2]
        wi = wi_ref[:, h * D:(h + 1) * D]
        ohs = [(hist_ref[h * B2:(h + 1) * B2, L + j:L + j + 1] == iota8)
               .astype(jnp.float32) for j in range(L)]
        oh = jnp.concatenate(ohs, axis=0)
        u_att = mm(wu, a1b_ref[:]) + ab1_ref[:]
        u_rep = jnp.broadcast_to(u_att[None], (L, B2, D)).reshape(LB2, D)
        x2 = relu(mm(wi, w1a_ref[:]) + mm(oh, rtw1b) + b1_ref[:])
        hh = relu(mm(x2, a1a_ref[:]) + u_rep)
        hh = relu(mm(hh, a2w_ref[:]) + a2b_ref[:])
        l2 = jnp.sum(hh * a3_ref[:], axis=1, keepdims=True)
        l3 = l2.reshape(L, B2, 1)
        m = l3[0]
        for j in range(1, L):
            m = jnp.maximum(m, l3[j])
        e3 = jnp.exp(l3 - m[None])
        denom = e3[0]
        for j in range(1, L):
            denom = denom + e3[j]
        w3 = x2.reshape(L, B2, D) * e3
        acc = w3[0]
        for j in range(1, L):
            acc = acc + w3[j]
        hI = acc / denom
        hI = relu(mm(hI, w2_ref[:]) + b2_ref[:])
        out_ref[h * B2:(h + 1) * B2] = relu(
            mm(wu, w3a_ref[:]) + mm(hI, w3b_ref[:]) + b3_ref[:])


def _stage_a(witem_w, hist_g, wuser, rt_pad, w1a, w1b, b1,
             a1a, a1b, ab1, a2w, a2b, a3row, w2, b2, w3a, w3b, b3,
             BU, blk0, nblk):
    N, D = wuser.shape
    Dh = hist_g.shape[1]
    L = 2 * witem_w.shape[0] // (nblk * BU)
    full = lambda arr: pl.BlockSpec(arr.shape, lambda i: (0,) * arr.ndim)
    return pl.pallas_call(
        _stage_a_body,
        grid=(nblk,),
        in_specs=[
            pl.BlockSpec((L * BU // 2, 2 * D), lambda i: (i, 0)),
            pl.BlockSpec((BU, Dh), lambda i: (i + blk0, 0)),
            pl.BlockSpec((BU, D), lambda i: (i + blk0, 0)),
            full(rt_pad), full(w1a), full(w1b), full(b1),
            full(a1a), full(a1b), full(ab1), full(a2w), full(a2b), full(a3row),
            full(w2), full(b2), full(w3a), full(w3b), full(b3),
        ],
        out_specs=pl.BlockSpec((BU, D), lambda i: (i, 0)),
        out_shape=jax.ShapeDtypeStruct((nblk * BU, D), jnp.float32),
    )(witem_w, hist_g, wuser, rt_pad, w1a, w1b, b1,
      a1a, a1b, ab1, a2w, a2b, a3row, w2, b2, w3a, w3b, b3)


def _stage_b_body(f_ref, wu_ref,
                  sa1a_ref, sa1b_ref, sab1_ref, sa2w_ref, sa2b_ref, sa3_ref,
                  s1w_ref, s1b_ref, s2a_ref, s2b_ref, s2bias_ref,
                  s3w_ref, s3b_ref, out_ref):
    S = f_ref.shape[0] - 1
    relu = lambda x: jnp.maximum(x, 0.0)
    mm = lambda a, b: jnp.dot(a.astype(jnp.bfloat16), b.astype(jnp.bfloat16),
                              preferred_element_type=jnp.float32)
    wu = wu_ref[:]
    u_att = mm(wu, sa1b_ref[:]) + sab1_ref[:]
    sa2b = sa2b_ref[:]
    sa3 = sa3_ref[:]
    zs = []
    ls = []
    for s in range(S):
        z_s = f_ref[s + 1]
        a = relu(mm(z_s, sa1a_ref[:]) + u_att)
        a = relu(mm(a, sa2w_ref[:]) + sa2b)
        l_s = jnp.sum(a * sa3, axis=1, keepdims=True)
        zs.append(z_s)
        ls.append(l_s)
    m = ls[0]
    for s in range(1, S):
        m = jnp.maximum(m, ls[s])
    es = [jnp.exp(l_s - m) for l_s in ls]
    denom = es[0]
    for s in range(1, S):
        denom = denom + es[s]
    acc = zs[0] * es[0]
    for s in range(1, S):
        acc = acc + zs[s] * es[s]
    hS = acc / denom
    hS = relu(mm(hS, s1w_ref[:]) + s1b_ref[:])
    f = relu(mm(f_ref[0], s2a_ref[:]) + mm(hS, s2b_ref[:]) + s2bias_ref[:])
    out_ref[:] = relu(mm(f, s3w_ref[:]) + s3b_ref[:])


def _stage_b(feat3, wuser, sa1a, sa1b, sab1, sa2w, sa2b, sa3row,
             s1w, s1b, s2a, s2b, s2bias, s3w, s3b):
    S1, B, D = feat3.shape
    BN = 512
    grid = B // BN
    full = lambda arr: pl.BlockSpec(arr.shape, lambda i: (0,) * arr.ndim)
    return pl.pallas_call(
        _stage_b_body,
        grid=(grid,),
        in_specs=[
            pl.BlockSpec((S1, BN, D), lambda i: (0, i, 0)),
            pl.BlockSpec((BN, D), lambda i: (i, 0)),
            full(sa1a), full(sa1b), full(sab1), full(sa2w), full(sa2b),
            full(sa3row), full(s1w), full(s1b), full(s2a), full(s2b),
            full(s2bias), full(s3w), full(s3b),
        ],
        out_specs=pl.BlockSpec((BN, D), lambda i: (i, 0)),
        out_shape=jax.ShapeDtypeStruct((B, D), jnp.float32),
    )(feat3, wuser, sa1a, sa1b, sab1, sa2w, sa2b, sa3row,
      s1w, s1b, s2a, s2b, s2bias, s3w, s3b)


def kernel(nodes, item_history, itemrating_history, social_history,
           user_table, item_table, rating_table,
           i_ln1_w, i_ln1_b, i_ln2_w, i_ln2_b, i_ln3_w, i_ln3_b,
           ia1_w, ia1_b, ia2_w, ia2_b, ia3_w, ia3_b,
           s_ln1_w, s_ln1_b, s_ln2_w, s_ln2_b, s_ln3_w, s_ln3_b,
           sa1_w, sa1_b, sa2_w, sa2_b, sa3_w, sa3_b):
    NU, L = item_history.shape
    S = social_history.shape[1]
    D = user_table.shape[1]
    B = nodes.shape[0]
    i32 = jnp.int32
    f32 = jnp.float32
    nodes = nodes.astype(i32)

    cat_all = jnp.concatenate(
        [item_history.astype(i32), itemrating_history.astype(i32),
         social_history.astype(i32), jnp.zeros((NU, 4), i32)], axis=1)

    soc = _gather_rows(cat_all, nodes, 32)[:, 2 * L:2 * L + S]
    u_all = jnp.concatenate([nodes, soc.T.reshape(-1)])
    NT = u_all.shape[0]

    hist_g, wuser_g = _gather_rows2(cat_all, user_table.astype(f32),
                                    u_all, 112)

    BU = 512
    NB = NT // BU
    B2 = BU // 2
    items_perm = (hist_g[:, :L].reshape(NB, 2, B2, L)
                  .transpose(0, 3, 2, 1).reshape(-1))
    itemtab = item_table.astype(f32)
    NH = NB // 2
    nper = L * NT // 2
    wi_halves = [
        _gather_rows_ring(itemtab, items_perm[h * nper:(h + 1) * nper], 120)
        .reshape(nper // 2, 2 * D)
        for h in (0, 1)
    ]

    rt_pad = jnp.concatenate(
        [rating_table.astype(f32),
         jnp.zeros((8 - rating_table.shape[0], D), f32)], axis=0)
    row = lambda v: v.reshape(1, -1).astype(f32)
    feat_halves = [
        _stage_a(
            wi_halves[h], hist_g, wuser_g, rt_pad,
            i_ln1_w[:D], i_ln1_w[D:], row(i_ln1_b),
            ia1_w[:D], ia1_w[D:], row(ia1_b), ia2_w, row(ia2_b), row(ia3_w),
            i_ln2_w, row(i_ln2_b), i_ln3_w[:D], i_ln3_w[D:], row(i_ln3_b),
            BU, h * NH, NH)
        for h in (0, 1)
    ]
    feat = jnp.concatenate(feat_halves, axis=0)

    feat3 = feat.reshape(S + 1, B, D)
    return _stage_b(
        feat3, wuser_g,
        sa1_w[:D], sa1_w[D:], row(sa1_b), sa2_w, row(sa2_b), row(sa3_w),
        s_ln1_w, row(s_ln1_b), s_ln2_w[:D], s_ln2_w[D:], row(s_ln2_b),
        s_ln3_w, row(s_ln3_b))
